# baseline (speedup 1.0000x reference)
import functools

import jax
import jax.numpy as jnp
from jax import lax
from jax.experimental import pallas as pl
from jax.experimental.pallas import tpu as pltpu
from jax.experimental.pallas import tpu_sc as plsc

N = 10000
E = 320000
D = 128

NC = 2
NS = 16
NW = NC * NS
BLK = 128
NBLK = 80
EPW = NBLK * BLK
E_PAD = NW * EPW
N_ACC = 10112
RPS = N_ACC // NS


_MM_ROWS = 1000


def _mm_body(x_ref, wt_ref, b_ref, o_ref):
    o_ref[...] = (
        jnp.dot(
            x_ref[...],
            wt_ref[...],
            preferred_element_type=jnp.float32,
            precision=lax.Precision.HIGHEST,
        )
        + b_ref[...]
    )


def _linear(x2d, W, b):
    wt = W.T
    b2 = b.reshape(1, D)
    return pl.pallas_call(
        _mm_body,
        grid=(N // _MM_ROWS,),
        in_specs=[
            pl.BlockSpec((_MM_ROWS, D), lambda i: (i, 0)),
            pl.BlockSpec((D, D), lambda i: (0, 0)),
            pl.BlockSpec((1, D), lambda i: (0, 0)),
        ],
        out_specs=pl.BlockSpec((_MM_ROWS, D), lambda i: (i, 0)),
        out_shape=jax.ShapeDtypeStruct((N, D), jnp.float32),
    )(x2d, wt, b2)


_sc_mesh = plsc.VectorSubcoreMesh(core_axis_name="c", subcore_axis_name="s")


NBUF = 2
NCHUNK = 2
CHB = NBLK // NCHUNK


@functools.partial(
    pl.kernel,
    mesh=_sc_mesh,
    out_type=jax.ShapeDtypeStruct((NC, N_ACC, D), jnp.float32),
    scratch_types=[
        pltpu.VMEM((CHB, BLK), jnp.int32),
        pltpu.VMEM((CHB, BLK), jnp.int32),
        pltpu.VMEM((NBUF, BLK, D), jnp.float32),
        pltpu.VMEM_SHARED((N_ACC, D), jnp.float32),
    ]
    + [pltpu.SemaphoreType.DMA] * (2 * NBUF),
)
def _sc_spmm(h_hbm, src_hbm, dst_hbm, zero_hbm, out_hbm, src_v, dst_v, rows_v, acc, *sems):
    c = lax.axis_index("c")
    s = lax.axis_index("s")
    wid = s * NC + c

    pltpu.sync_copy(zero_hbm.at[pl.ds(s * RPS, RPS)], acc.at[pl.ds(s * RPS, RPS)])

    plsc.subcore_barrier()

    @pl.loop(0, NCHUNK)
    def _(ci):
        base = ci * CHB
        pltpu.sync_copy(src_hbm.at[wid, pl.ds(base, CHB)], src_v)
        pltpu.sync_copy(dst_hbm.at[wid, pl.ds(base, CHB)], dst_v)

        for k in range(NBUF):
            pltpu.async_copy(h_hbm.at[src_v.at[k]], rows_v.at[k], sems[k])

        @pl.loop(0, CHB, step=NBUF)
        def _(j):
            for k in range(NBUF):
                pltpu.make_async_copy(
                    h_hbm.at[src_v.at[0]], rows_v.at[k], sems[k]
                ).wait()
                pltpu.async_copy(
                    rows_v.at[k], acc.at[dst_v.at[j + k]], sems[NBUF + k], add=True
                )
            for k in range(NBUF):
                pltpu.make_async_copy(
                    rows_v.at[k], acc.at[dst_v.at[0]], sems[NBUF + k]
                ).wait()

                @pl.when(j + NBUF + k < CHB)
                def _():
                    pltpu.async_copy(
                        h_hbm.at[src_v.at[j + NBUF + k]], rows_v.at[k], sems[k]
                    )

    plsc.subcore_barrier()

    pltpu.sync_copy(
        acc.at[pl.ds(s * RPS, RPS)], out_hbm.at[c, pl.ds(s * RPS, RPS)]
    )


def _fin_body(p_ref, a_ref, o_ref):
    t = p_ref[0] + p_ref[1]
    o_ref[0] = jnp.where(t >= 0.0, t, a_ref[0, 0] * t)


def _finish(partials, alpha):
    a2 = alpha.reshape(1, 1)
    return pl.pallas_call(
        _fin_body,
        grid=(N // _MM_ROWS,),
        in_specs=[
            pl.BlockSpec((NC, _MM_ROWS, D), lambda i: (0, i, 0)),
            pl.BlockSpec((1, 1), lambda i: (0, 0)),
        ],
        out_specs=pl.BlockSpec((1, _MM_ROWS, D), lambda i: (0, i, 0)),
        out_shape=jax.ShapeDtypeStruct((1, N, D), jnp.float32),
    )(partials, a2)


@jax.jit
def kernel(x, edge_index, W, b, alpha):
    h = _linear(x[0], W, b)

    dst = edge_index[0]
    src = edge_index[1]
    pad = E_PAD - E
    pad_i = jnp.arange(pad, dtype=jnp.int32)
    pad_src = (pad_i * 37) % N
    pad_dst = N + (pad_i % (N_ACC - N))
    src_p = jnp.concatenate([src, pad_src]).reshape(NW, NBLK, BLK)
    dst_p = jnp.concatenate([dst, pad_dst]).reshape(NW, NBLK, BLK)

    zero = jnp.zeros((N_ACC, D), jnp.float32)
    partials = _sc_spmm(h, src_p, dst_p, zero)

    return _finish(partials, alpha)


# speedup vs baseline: 1.2194x; 1.2194x over previous
import functools

import jax
import jax.numpy as jnp
from jax import lax
from jax.experimental import pallas as pl
from jax.experimental.pallas import tpu as pltpu
from jax.experimental.pallas import tpu_sc as plsc

N = 10000
E = 320000
D = 128

NC = 2
NS = 16
NW = NC * NS
BLK = 128
NBLK = 80
EPW = NBLK * BLK
E_PAD = NW * EPW
N_ACC = 10112
RPS = N_ACC // NS


_MM_ROWS = 1000


def _mm_body(x_ref, wt_ref, b_ref, o_ref):
    o_ref[...] = (
        jnp.dot(
            x_ref[...],
            wt_ref[...],
            preferred_element_type=jnp.float32,
            precision=lax.Precision.HIGHEST,
        )
        + b_ref[...]
    )


def _linear(x2d, W, b):
    wt = W.T
    b2 = b.reshape(1, D)
    return pl.pallas_call(
        _mm_body,
        grid=(N // _MM_ROWS,),
        in_specs=[
            pl.BlockSpec((_MM_ROWS, D), lambda i: (i, 0)),
            pl.BlockSpec((D, D), lambda i: (0, 0)),
            pl.BlockSpec((1, D), lambda i: (0, 0)),
        ],
        out_specs=pl.BlockSpec((_MM_ROWS, D), lambda i: (i, 0)),
        out_shape=jax.ShapeDtypeStruct((N, D), jnp.float32),
    )(x2d, wt, b2)


_sc_mesh = plsc.VectorSubcoreMesh(core_axis_name="c", subcore_axis_name="s")


NBUF = 2
NCHUNK = 2
CHB = NBLK // NCHUNK


@functools.partial(
    pl.kernel,
    mesh=_sc_mesh,
    out_type=jax.ShapeDtypeStruct((NC, N_ACC, D), jnp.float32),
    scratch_types=[
        pltpu.VMEM((CHB, BLK), jnp.int32),
        pltpu.VMEM((CHB, BLK), jnp.int32),
        pltpu.VMEM((NBUF, BLK, D), jnp.float32),
        pltpu.VMEM_SHARED((N_ACC, D), jnp.float32),
    ]
    + [pltpu.SemaphoreType.DMA] * (2 * NBUF),
)
def _sc_spmm(h_hbm, src_hbm, dst_hbm, zero_hbm, out_hbm, src_v, dst_v, rows_v, acc, *sems):
    c = lax.axis_index("c")
    s = lax.axis_index("s")
    wid = s * NC + c

    pltpu.sync_copy(zero_hbm.at[pl.ds(s * RPS, RPS)], acc.at[pl.ds(s * RPS, RPS)])

    plsc.subcore_barrier()

    @pl.loop(0, NCHUNK)
    def _(ci):
        base = ci * CHB
        pltpu.sync_copy(src_hbm.at[wid, pl.ds(base, CHB)], src_v)
        pltpu.sync_copy(dst_hbm.at[wid, pl.ds(base, CHB)], dst_v)

        for k in range(NBUF):
            pltpu.async_copy(h_hbm.at[src_v.at[k]], rows_v.at[k], sems[k])

        @pl.loop(0, CHB, step=NBUF)
        def _(j):
            for k in range(NBUF):
                pltpu.make_async_copy(
                    h_hbm.at[src_v.at[0]], rows_v.at[k], sems[k]
                ).wait()
                pltpu.sync_copy(rows_v.at[k], acc.at[dst_v.at[j + k]], add=True)

                @pl.when(j + NBUF + k < CHB)
                def _():
                    pltpu.async_copy(
                        h_hbm.at[src_v.at[j + NBUF + k]], rows_v.at[k], sems[k]
                    )

    plsc.subcore_barrier()

    pltpu.sync_copy(
        acc.at[pl.ds(s * RPS, RPS)], out_hbm.at[c, pl.ds(s * RPS, RPS)]
    )


def _fin_body(p_ref, a_ref, o_ref):
    t = p_ref[0] + p_ref[1]
    o_ref[0] = jnp.where(t >= 0.0, t, a_ref[0, 0] * t)


def _finish(partials, alpha):
    a2 = alpha.reshape(1, 1)
    return pl.pallas_call(
        _fin_body,
        grid=(N // _MM_ROWS,),
        in_specs=[
            pl.BlockSpec((NC, _MM_ROWS, D), lambda i: (0, i, 0)),
            pl.BlockSpec((1, 1), lambda i: (0, 0)),
        ],
        out_specs=pl.BlockSpec((1, _MM_ROWS, D), lambda i: (0, i, 0)),
        out_shape=jax.ShapeDtypeStruct((1, N, D), jnp.float32),
    )(partials, a2)


@jax.jit
def kernel(x, edge_index, W, b, alpha):
    h = _linear(x[0], W, b)

    dst = edge_index[0]
    src = edge_index[1]
    pad = E_PAD - E
    pad_i = jnp.arange(pad, dtype=jnp.int32)
    pad_src = (pad_i * 37) % N
    pad_dst = N + (pad_i % (N_ACC - N))
    src_p = jnp.concatenate([src, pad_src]).reshape(NW, NBLK, BLK)
    dst_p = jnp.concatenate([dst, pad_dst]).reshape(NW, NBLK, BLK)

    zero = jnp.zeros((N_ACC, D), jnp.float32)
    partials = _sc_spmm(h, src_p, dst_p, zero)

    return _finish(partials, alpha)


# speedup vs baseline: 1.2353x; 1.0130x over previous
import functools

import jax
import jax.numpy as jnp
from jax import lax
from jax.experimental import pallas as pl
from jax.experimental.pallas import tpu as pltpu
from jax.experimental.pallas import tpu_sc as plsc

N = 10000
E = 320000
D = 128

NC = 2
NS = 16
NW = NC * NS
BLK = 112
NBLK = 90
EPW = NBLK * BLK
E_PAD = NW * EPW
N_ACC = 10112
RPS = N_ACC // NS


_MM_ROWS = 1000


def _mm_body(x_ref, wt_ref, b_ref, o_ref):
    o_ref[...] = (
        jnp.dot(
            x_ref[...],
            wt_ref[...],
            preferred_element_type=jnp.float32,
            precision=lax.Precision.HIGHEST,
        )
        + b_ref[...]
    )


def _linear(x2d, W, b):
    wt = W.T
    b2 = b.reshape(1, D)
    return pl.pallas_call(
        _mm_body,
        grid=(N // _MM_ROWS,),
        in_specs=[
            pl.BlockSpec((_MM_ROWS, D), lambda i: (i, 0)),
            pl.BlockSpec((D, D), lambda i: (0, 0)),
            pl.BlockSpec((1, D), lambda i: (0, 0)),
        ],
        out_specs=pl.BlockSpec((_MM_ROWS, D), lambda i: (i, 0)),
        out_shape=jax.ShapeDtypeStruct((N, D), jnp.float32),
    )(x2d, wt, b2)


_sc_mesh = plsc.VectorSubcoreMesh(core_axis_name="c", subcore_axis_name="s")


NBUF = 3
NCHUNK = 5
CHB = NBLK // NCHUNK


@functools.partial(
    pl.kernel,
    mesh=_sc_mesh,
    out_type=jax.ShapeDtypeStruct((NC, N_ACC, D), jnp.float32),
    scratch_types=[
        pltpu.VMEM((CHB, BLK), jnp.int32),
        pltpu.VMEM((CHB, BLK), jnp.int32),
        pltpu.VMEM((NBUF, BLK, D), jnp.float32),
        pltpu.VMEM_SHARED((N_ACC, D), jnp.float32),
    ]
    + [pltpu.SemaphoreType.DMA] * (NBUF + 1),
)
def _sc_spmm(h_hbm, src_hbm, dst_hbm, zero_hbm, out_hbm, src_v, dst_v, rows_v, acc, *sems):
    c = lax.axis_index("c")
    s = lax.axis_index("s")
    wid = s * NC + c

    pltpu.async_copy(
        zero_hbm.at[pl.ds(s * RPS, RPS)], acc.at[pl.ds(s * RPS, RPS)], sems[NBUF]
    )

    @pl.loop(0, NCHUNK)
    def _(ci):
        pltpu.sync_copy(src_hbm.at[wid, ci], src_v)
        pltpu.sync_copy(dst_hbm.at[wid, ci], dst_v)

        for k in range(NBUF):
            pltpu.async_copy(h_hbm.at[src_v.at[k]], rows_v.at[k], sems[k])

        @pl.when(ci == 0)
        def _():
            pltpu.make_async_copy(
                zero_hbm.at[pl.ds(s * RPS, RPS)],
                acc.at[pl.ds(s * RPS, RPS)],
                sems[NBUF],
            ).wait()
            plsc.subcore_barrier()

        @pl.loop(0, CHB - NBUF, step=NBUF)
        def _(j):
            for k in range(NBUF):
                pltpu.make_async_copy(
                    h_hbm.at[src_v.at[0]], rows_v.at[k], sems[k]
                ).wait()
                pltpu.sync_copy(rows_v.at[k], acc.at[dst_v.at[j + k]], add=True)
                pltpu.async_copy(
                    h_hbm.at[src_v.at[j + NBUF + k]], rows_v.at[k], sems[k]
                )

        for k in range(NBUF):
            pltpu.make_async_copy(h_hbm.at[src_v.at[0]], rows_v.at[k], sems[k]).wait()
            pltpu.sync_copy(
                rows_v.at[k], acc.at[dst_v.at[CHB - NBUF + k]], add=True
            )

    plsc.subcore_barrier()

    pltpu.sync_copy(
        acc.at[pl.ds(s * RPS, RPS)], out_hbm.at[c, pl.ds(s * RPS, RPS)]
    )


def _fin_body(p_ref, a_ref, o_ref):
    t = p_ref[0] + p_ref[1]
    o_ref[0] = jnp.where(t >= 0.0, t, a_ref[0, 0] * t)


def _finish(partials, alpha):
    a2 = alpha.reshape(1, 1)
    return pl.pallas_call(
        _fin_body,
        grid=(N // _MM_ROWS,),
        in_specs=[
            pl.BlockSpec((NC, _MM_ROWS, D), lambda i: (0, i, 0)),
            pl.BlockSpec((1, 1), lambda i: (0, 0)),
        ],
        out_specs=pl.BlockSpec((1, _MM_ROWS, D), lambda i: (0, i, 0)),
        out_shape=jax.ShapeDtypeStruct((1, N, D), jnp.float32),
    )(partials, a2)


@jax.jit
def kernel(x, edge_index, W, b, alpha):
    h = _linear(x[0], W, b)

    dst = edge_index[0]
    src = edge_index[1]
    pad = E_PAD - E
    pad_i = jnp.arange(pad, dtype=jnp.int32)
    pad_src = (pad_i * 37) % N
    pad_dst = N + (pad_i % (N_ACC - N))
    src_p = jnp.concatenate([src, pad_src]).reshape(NW, NCHUNK, CHB, BLK)
    dst_p = jnp.concatenate([dst, pad_dst]).reshape(NW, NCHUNK, CHB, BLK)

    zero = jnp.zeros((N_ACC, D), jnp.float32)
    partials = _sc_spmm(h, src_p, dst_p, zero)

    return _finish(partials, alpha)
